# baseline (speedup 1.0000x reference)
import functools

import jax
import jax.numpy as jnp
from jax import lax
from jax.experimental import pallas as pl
from jax.experimental.pallas import tpu as pltpu
from jax.experimental.pallas import tpu_sc as plsc

D_EMBED = 64
MAX_LEN = 200
NUM_CORES = 2
NUM_SUBCORES = 16
NW = NUM_CORES * NUM_SUBCORES
CHUNK = 512
LANES = 16


def _sc_gather_body(idx_hbm, te_hbm, out_hbm, table_v, ibuf0, ibuf1, obuf0, obuf1, sem_i, sem_o):
    ibufs = (ibuf0, ibuf1)
    obufs = (obuf0, obuf1)
    total_rows = idx_hbm.shape[0]
    rows_per_w = total_rows // NW
    n_chunks = rows_per_w // CHUNK

    wid = lax.axis_index("s") * NUM_CORES + lax.axis_index("c")
    row_base = wid * rows_per_w

    pltpu.sync_copy(te_hbm, table_v)

    iota16 = lax.iota(jnp.int32, LANES)
    out_lane_base = iota16 * D_EMBED

    for b in range(2):
        pltpu.make_async_copy(
            idx_hbm.at[pl.ds(pl.multiple_of(row_base + b * CHUNK, CHUNK), CHUNK)],
            ibufs[b],
            sem_i.at[b],
        ).start()

    def _compute_chunk(ib, ob):
        def jbody(j, carry):
            rvec = ib[pl.ds(j * LANES, LANES)]
            raddr = rvec * D_EMBED
            oaddr = j * (LANES * D_EMBED) + out_lane_base
            for d in range(D_EMBED):
                vals = plsc.load_gather(table_v, [raddr + d])
                plsc.store_scatter(ob, [oaddr + d], vals)
            return carry

        lax.fori_loop(0, CHUNK // LANES, jbody, 0)

    def outer(g, carry):
        for b in range(2):
            cg = g * 2 + b
            pltpu.make_async_copy(
                idx_hbm.at[pl.ds(0, CHUNK)], ibufs[b], sem_i.at[b]
            ).wait()
            @pl.when(g > 0)
            def _wait_out():
                pltpu.make_async_copy(
                    obufs[b], out_hbm.at[pl.ds(0, CHUNK * D_EMBED)], sem_o.at[b]
                ).wait()

            _compute_chunk(ibufs[b], obufs[b])

            @pl.when(cg + 2 < n_chunks)
            def _prefetch():
                nstart = pl.multiple_of(row_base + (cg + 2) * CHUNK, CHUNK)
                pltpu.make_async_copy(
                    idx_hbm.at[pl.ds(nstart, CHUNK)], ibufs[b], sem_i.at[b]
                ).start()

            ostart = pl.multiple_of(
                (row_base + cg * CHUNK) * D_EMBED, CHUNK * D_EMBED
            )
            pltpu.make_async_copy(
                obufs[b],
                out_hbm.at[pl.ds(ostart, CHUNK * D_EMBED)],
                sem_o.at[b],
            ).start()
        return carry

    lax.fori_loop(0, n_chunks // 2, outer, 0)

    for b in range(2):
        pltpu.make_async_copy(
            obufs[b], out_hbm.at[pl.ds(0, CHUNK * D_EMBED)], sem_o.at[b]
        ).wait()


@jax.jit
def kernel(time_idxs, te):
    batch, hist = time_idxs.shape
    total_rows = batch * hist
    idx_flat = time_idxs.reshape(total_rows)
    te_flat = te.reshape(MAX_LEN * D_EMBED)
    mesh = plsc.VectorSubcoreMesh(
        core_axis_name="c", subcore_axis_name="s",
        num_cores=NUM_CORES, num_subcores=NUM_SUBCORES,
    )
    sc_call = functools.partial(
        pl.kernel,
        out_type=jax.ShapeDtypeStruct((total_rows * D_EMBED,), jnp.float32),
        mesh=mesh,
        scratch_types=[
            pltpu.VMEM((MAX_LEN * D_EMBED,), jnp.float32),
            pltpu.VMEM((CHUNK,), jnp.int32),
            pltpu.VMEM((CHUNK,), jnp.int32),
            pltpu.VMEM((CHUNK * D_EMBED,), jnp.float32),
            pltpu.VMEM((CHUNK * D_EMBED,), jnp.float32),
            pltpu.SemaphoreType.DMA((2,)),
            pltpu.SemaphoreType.DMA((2,)),
        ],
        compiler_params=pltpu.CompilerParams(needs_layout_passes=False),
    )(_sc_gather_body)
    out_flat = sc_call(idx_flat, te_flat)
    return out_flat.reshape(batch, hist, D_EMBED)


# speedup vs baseline: 2.1613x; 2.1613x over previous
import functools

import jax
import jax.numpy as jnp
from jax import lax
from jax.experimental import pallas as pl
from jax.experimental.pallas import tpu as pltpu
from jax.experimental.pallas import tpu_sc as plsc

D_EMBED = 64
MAX_LEN = 200
NUM_CORES = 2
NUM_SUBCORES = 16
NW = NUM_CORES * NUM_SUBCORES
CHUNK = 512
LANES = 16


def _sc_gather_body(idx_hbm, te_hbm, out_hbm, table_v, ibuf0, ibuf1, obuf0, obuf1, sem_i, sem_o):
    ibufs = (ibuf0, ibuf1)
    obufs = (obuf0, obuf1)
    total_rows = idx_hbm.shape[0]
    rows_per_w = total_rows // NW
    n_chunks = rows_per_w // CHUNK

    wid = lax.axis_index("s") * NUM_CORES + lax.axis_index("c")
    row_base = wid * rows_per_w

    pltpu.sync_copy(te_hbm, table_v)

    iota16 = lax.iota(jnp.int32, LANES)
    zeros16 = iota16 * 0

    for b in range(2):
        pltpu.make_async_copy(
            idx_hbm.at[pl.ds(pl.multiple_of(row_base + b * CHUNK, CHUNK), CHUNK)],
            ibufs[b],
            sem_i.at[b],
        ).start()

    def _compute_chunk(ib, ob):
        def rbody(r, carry):
            rsplat = plsc.load_gather(ib, [zeros16 + r])
            src = rsplat * D_EMBED + iota16
            for k in range(D_EMBED // LANES):
                vals = plsc.load_gather(table_v, [src + (k * LANES)])
                ob[pl.ds(r * D_EMBED + k * LANES, LANES)] = vals
            return carry

        lax.fori_loop(0, CHUNK, rbody, 0)

    def outer(g, carry):
        for b in range(2):
            cg = g * 2 + b
            pltpu.make_async_copy(
                idx_hbm.at[pl.ds(0, CHUNK)], ibufs[b], sem_i.at[b]
            ).wait()
            @pl.when(g > 0)
            def _wait_out():
                pltpu.make_async_copy(
                    obufs[b], out_hbm.at[pl.ds(0, CHUNK * D_EMBED)], sem_o.at[b]
                ).wait()

            _compute_chunk(ibufs[b], obufs[b])

            @pl.when(cg + 2 < n_chunks)
            def _prefetch():
                nstart = pl.multiple_of(row_base + (cg + 2) * CHUNK, CHUNK)
                pltpu.make_async_copy(
                    idx_hbm.at[pl.ds(nstart, CHUNK)], ibufs[b], sem_i.at[b]
                ).start()

            ostart = pl.multiple_of(
                (row_base + cg * CHUNK) * D_EMBED, CHUNK * D_EMBED
            )
            pltpu.make_async_copy(
                obufs[b],
                out_hbm.at[pl.ds(ostart, CHUNK * D_EMBED)],
                sem_o.at[b],
            ).start()
        return carry

    lax.fori_loop(0, n_chunks // 2, outer, 0)

    for b in range(2):
        pltpu.make_async_copy(
            obufs[b], out_hbm.at[pl.ds(0, CHUNK * D_EMBED)], sem_o.at[b]
        ).wait()


@jax.jit
def kernel(time_idxs, te):
    batch, hist = time_idxs.shape
    total_rows = batch * hist
    idx_flat = time_idxs.reshape(total_rows)
    te_flat = te.reshape(MAX_LEN * D_EMBED)
    mesh = plsc.VectorSubcoreMesh(
        core_axis_name="c", subcore_axis_name="s",
        num_cores=NUM_CORES, num_subcores=NUM_SUBCORES,
    )
    sc_call = functools.partial(
        pl.kernel,
        out_type=jax.ShapeDtypeStruct((total_rows * D_EMBED,), jnp.float32),
        mesh=mesh,
        scratch_types=[
            pltpu.VMEM((MAX_LEN * D_EMBED,), jnp.float32),
            pltpu.VMEM((CHUNK,), jnp.int32),
            pltpu.VMEM((CHUNK,), jnp.int32),
            pltpu.VMEM((CHUNK * D_EMBED,), jnp.float32),
            pltpu.VMEM((CHUNK * D_EMBED,), jnp.float32),
            pltpu.SemaphoreType.DMA((2,)),
            pltpu.SemaphoreType.DMA((2,)),
        ],
        compiler_params=pltpu.CompilerParams(needs_layout_passes=False),
    )(_sc_gather_body)
    out_flat = sc_call(idx_flat, te_flat)
    return out_flat.reshape(batch, hist, D_EMBED)


# speedup vs baseline: 4.1443x; 1.9175x over previous
import functools

import jax
import jax.numpy as jnp
from jax import lax
from jax.experimental import pallas as pl
from jax.experimental.pallas import tpu as pltpu
from jax.experimental.pallas import tpu_sc as plsc

D_EMBED = 64
MAX_LEN = 200
NUM_CORES = 2
NUM_SUBCORES = 16
NW = NUM_CORES * NUM_SUBCORES
CHUNK = 512
LANES = 16


def _sc_gather_body(idx_hbm, te_hbm, out_hbm, table_v, ibuf0, ibuf1, obuf0, obuf1, sem_i, sem_o):
    ibufs = (ibuf0, ibuf1)
    obufs = (obuf0, obuf1)
    total_rows = idx_hbm.shape[0]
    rows_per_w = total_rows // NW
    n_chunks = rows_per_w // CHUNK

    wid = lax.axis_index("s") * NUM_CORES + lax.axis_index("c")
    row_base = wid * rows_per_w

    pltpu.sync_copy(te_hbm, table_v)

    iota16 = lax.iota(jnp.int32, LANES)
    zeros16 = iota16 * 0

    for b in range(2):
        pltpu.make_async_copy(
            idx_hbm.at[pl.ds(pl.multiple_of(row_base + b * CHUNK, CHUNK), CHUNK)],
            ibufs[b],
            sem_i.at[b],
        ).start()

    def _compute_chunk(ib, ob):
        @plsc.parallel_loop(0, CHUNK, step=1, unroll=8)
        def rbody(r):
            rsplat = plsc.load_gather(ib, [zeros16 + r])
            src = rsplat * D_EMBED + iota16
            for k in range(D_EMBED // LANES):
                vals = plsc.load_gather(table_v, [src + (k * LANES)])
                ob[pl.ds(r * D_EMBED + k * LANES, LANES)] = vals

    def outer(g, carry):
        for b in range(2):
            cg = g * 2 + b
            pltpu.make_async_copy(
                idx_hbm.at[pl.ds(0, CHUNK)], ibufs[b], sem_i.at[b]
            ).wait()
            @pl.when(g > 0)
            def _wait_out():
                pltpu.make_async_copy(
                    obufs[b], out_hbm.at[pl.ds(0, CHUNK * D_EMBED)], sem_o.at[b]
                ).wait()

            _compute_chunk(ibufs[b], obufs[b])

            @pl.when(cg + 2 < n_chunks)
            def _prefetch():
                nstart = pl.multiple_of(row_base + (cg + 2) * CHUNK, CHUNK)
                pltpu.make_async_copy(
                    idx_hbm.at[pl.ds(nstart, CHUNK)], ibufs[b], sem_i.at[b]
                ).start()

            ostart = pl.multiple_of(
                (row_base + cg * CHUNK) * D_EMBED, CHUNK * D_EMBED
            )
            pltpu.make_async_copy(
                obufs[b],
                out_hbm.at[pl.ds(ostart, CHUNK * D_EMBED)],
                sem_o.at[b],
            ).start()
        return carry

    lax.fori_loop(0, n_chunks // 2, outer, 0)

    for b in range(2):
        pltpu.make_async_copy(
            obufs[b], out_hbm.at[pl.ds(0, CHUNK * D_EMBED)], sem_o.at[b]
        ).wait()


@jax.jit
def kernel(time_idxs, te):
    batch, hist = time_idxs.shape
    total_rows = batch * hist
    idx_flat = time_idxs.reshape(total_rows)
    te_flat = te.reshape(MAX_LEN * D_EMBED)
    mesh = plsc.VectorSubcoreMesh(
        core_axis_name="c", subcore_axis_name="s",
        num_cores=NUM_CORES, num_subcores=NUM_SUBCORES,
    )
    sc_call = functools.partial(
        pl.kernel,
        out_type=jax.ShapeDtypeStruct((total_rows * D_EMBED,), jnp.float32),
        mesh=mesh,
        scratch_types=[
            pltpu.VMEM((MAX_LEN * D_EMBED,), jnp.float32),
            pltpu.VMEM((CHUNK,), jnp.int32),
            pltpu.VMEM((CHUNK,), jnp.int32),
            pltpu.VMEM((CHUNK * D_EMBED,), jnp.float32),
            pltpu.VMEM((CHUNK * D_EMBED,), jnp.float32),
            pltpu.SemaphoreType.DMA((2,)),
            pltpu.SemaphoreType.DMA((2,)),
        ],
        compiler_params=pltpu.CompilerParams(needs_layout_passes=False),
    )(_sc_gather_body)
    out_flat = sc_call(idx_flat, te_flat)
    return out_flat.reshape(batch, hist, D_EMBED)
